# baseline (speedup 1.0000x reference)
import functools

import jax
import jax.numpy as jnp
from jax import lax
from jax.experimental import pallas as pl
from jax.experimental.pallas import tpu as pltpu
from jax.experimental.pallas import tpu_sc as plsc

D_MODEL = 1024
D_FF = 4096
E = 8
TOP_K = 2
T = 2048
CAPACITY = 640
NSLOT = E * CAPACITY
TK = T * TOP_K

NW = 32
SPT = NSLOT // NW
TPT = T // NW

RBLK = 128
NRB = T // RBLK

FBLK = 2048
NFB = D_FF // FBLK


def _router_body(x_ref, wg_ref, slotd_ref, slotc_ref, gate_ref, carry_ref):
    i = pl.program_id(0)

    @pl.when(i == 0)
    def _():
        carry_ref[...] = jnp.zeros_like(carry_ref)

    xb = x_ref[...]
    wg = wg_ref[...]
    logits = jnp.dot(xb, wg, preferred_element_type=jnp.float32)
    probs = jax.nn.softmax(logits, axis=-1)

    ids = lax.broadcasted_iota(jnp.int32, (RBLK, E), 1)
    m0 = jnp.max(probs, axis=-1, keepdims=True)
    am0 = jnp.min(jnp.where(probs == m0, ids, E), axis=-1, keepdims=True)
    probs2 = jnp.where(ids == am0, -1.0, probs)
    m1 = jnp.max(probs2, axis=-1, keepdims=True)
    am1 = jnp.min(jnp.where(probs2 == m1, ids, E), axis=-1, keepdims=True)
    gsum = m0 + m1 + 1e-9
    g0 = m0 / gsum
    g1 = m1 / gsum

    oh0 = (ids == am0).astype(jnp.float32)
    oh1 = (ids == am1).astype(jnp.float32)
    cnt = oh0 + oh1

    r = lax.broadcasted_iota(jnp.int32, (RBLK, RBLK), 0)
    c = lax.broadcasted_iota(jnp.int32, (RBLK, RBLK), 1)
    tri = (r > c).astype(jnp.float32)
    base = jnp.dot(tri, cnt, preferred_element_type=jnp.float32)
    base = base + carry_ref[...]
    carry_ref[...] = carry_ref[...] + jnp.sum(cnt, axis=0, keepdims=True)

    pos0 = jnp.sum(base * oh0, axis=-1, keepdims=True).astype(jnp.int32)
    pos1 = jnp.sum(base * oh1, axis=-1, keepdims=True).astype(jnp.int32)
    keep0 = pos0 < CAPACITY
    keep1 = pos1 < CAPACITY
    slot0 = am0 * CAPACITY + pos0
    slot1 = am1 * CAPACITY + pos1

    slotd_ref[...] = jnp.concatenate(
        [jnp.where(keep0, slot0, -1), jnp.where(keep1, slot1, -1)], axis=1)
    slotc_ref[...] = jnp.concatenate(
        [jnp.where(keep0, slot0, 0), jnp.where(keep1, slot1, 0)], axis=1)
    gate_ref[...] = jnp.concatenate(
        [g0 * keep0.astype(jnp.float32), g1 * keep1.astype(jnp.float32)], axis=1)


def _router(x, Wg):
    return pl.pallas_call(
        _router_body,
        grid=(NRB,),
        in_specs=[
            pl.BlockSpec((RBLK, D_MODEL), lambda i: (i, 0)),
            pl.BlockSpec((D_MODEL, E), lambda i: (0, 0)),
        ],
        out_specs=[
            pl.BlockSpec((RBLK, TOP_K), lambda i: (i, 0)),
            pl.BlockSpec((RBLK, TOP_K), lambda i: (i, 0)),
            pl.BlockSpec((RBLK, TOP_K), lambda i: (i, 0)),
        ],
        out_shape=[
            jax.ShapeDtypeStruct((T, TOP_K), jnp.int32),
            jax.ShapeDtypeStruct((T, TOP_K), jnp.int32),
            jax.ShapeDtypeStruct((T, TOP_K), jnp.float32),
        ],
        scratch_shapes=[pltpu.VMEM((1, E), jnp.float32)],
    )(x, Wg)


EPS = TK // 16


def _dispatch_body(slotd_hbm, x_hbm, out_hbm,
                   slot_v, idx_v, val_v, zero_v, src_v,
                   rows_a, rows_b, src_sp,
                   sem_g0, sem_g1, sem_w0, sem_w1, sem_s):
    cid = lax.axis_index("c")
    sid = lax.axis_index("s")
    wid = sid * 2 + cid
    lo = wid * SPT

    def z_i(i, _):
        zero_v[pl.ds(i * 16, 16)] = jnp.zeros((16,), jnp.int32)
        return 0
    lax.fori_loop(0, (NSLOT // 16) // 16, z_i, 0)
    pltpu.sync_copy(zero_v, src_sp.at[pl.ds(sid * (NSLOT // 16), NSLOT // 16)])

    cp_s = pltpu.async_copy(slotd_hbm.at[pl.ds(sid * EPS, EPS)], slot_v, sem_s)

    plsc.subcore_barrier()

    cp_s.wait()

    def chunk_j(j):
        def e_i(i, _):
            base = j * 128 + i * 16
            sv = slot_v[pl.ds(base, 16)]
            m = sv >= 0
            tvec = lax.shift_right_logical(
                sid * EPS + base + lax.iota(jnp.int32, 16), 1)
            idx_v[j, pl.ds(i * 16, 16)] = jnp.where(m, sv, 0)
            val_v[j, pl.ds(i * 16, 16)] = jnp.where(m, tvec, 0)
            return 0
        lax.fori_loop(0, 8, e_i, 0)

    for j in range(2):
        chunk_j(j)
        pltpu.sync_copy(val_v.at[j], src_sp.at[idx_v.at[j]], add=True)

    plsc.subcore_barrier()

    pltpu.sync_copy(src_sp.at[pl.ds(lo, SPT)], src_v)

    g0 = pltpu.async_copy(x_hbm.at[src_v.at[pl.ds(0, 80)]], rows_a, sem_g0)
    g1 = pltpu.async_copy(x_hbm.at[src_v.at[pl.ds(80, 80)]], rows_b, sem_g1)
    g0.wait()
    w0 = pltpu.async_copy(rows_a, out_hbm.at[pl.ds(lo, 80)], sem_w0)
    g1.wait()
    w1 = pltpu.async_copy(rows_b, out_hbm.at[pl.ds(lo + 80, 80)], sem_w1)
    w0.wait()
    w1.wait()


def _dispatch(slotd_flat, xbf):
    mesh = plsc.VectorSubcoreMesh(core_axis_name="c", subcore_axis_name="s", num_cores=2, num_subcores=16)
    return pl.kernel(
        _dispatch_body,
        out_type=jax.ShapeDtypeStruct((NSLOT, D_MODEL // 2), jnp.int32),
        mesh=mesh,
        compiler_params=pltpu.CompilerParams(needs_layout_passes=False),
        scratch_types=[
            pltpu.VMEM((EPS,), jnp.int32),
            pltpu.VMEM((2, 128), jnp.int32),
            pltpu.VMEM((2, 128), jnp.int32),
            pltpu.VMEM((NSLOT // 16,), jnp.int32),
            pltpu.VMEM((SPT,), jnp.int32),
            pltpu.VMEM((80, D_MODEL // 2), jnp.int32),
            pltpu.VMEM((80, D_MODEL // 2), jnp.int32),
            pltpu.VMEM_SHARED((NSLOT,), jnp.int32),
            pltpu.SemaphoreType.DMA,
            pltpu.SemaphoreType.DMA,
            pltpu.SemaphoreType.DMA,
            pltpu.SemaphoreType.DMA,
            pltpu.SemaphoreType.DMA,
        ],
    )(slotd_flat, xbf)


def _ffn_body(a_ref, w1_ref, b1_ref, w2_ref, b2_ref, y_ref, acc_ref):
    f = pl.program_id(1)
    a = a_ref[0]
    h = jnp.dot(a, w1_ref[0], preferred_element_type=jnp.float32)
    h = jnp.maximum(h + b1_ref[0], 0.0)
    hb = h.astype(jnp.bfloat16)
    part = jnp.dot(hb, w2_ref[0], preferred_element_type=jnp.float32)

    @pl.when(f == 0)
    def _():
        acc_ref[...] = part

    @pl.when(f != 0)
    def _():
        acc_ref[...] = acc_ref[...] + part

    @pl.when(f == NFB - 1)
    def _():
        y_ref[0] = acc_ref[...] + b2_ref[0]


def _ffn(bufs_bf, w1b, b1, w2b, b2):
    return pl.pallas_call(
        _ffn_body,
        grid=(E, NFB),
        in_specs=[
            pl.BlockSpec((1, CAPACITY, D_MODEL), lambda e, f: (e, 0, 0)),
            pl.BlockSpec((1, D_MODEL, FBLK), lambda e, f: (e, 0, f)),
            pl.BlockSpec((1, 1, FBLK), lambda e, f: (e, 0, f)),
            pl.BlockSpec((1, FBLK, D_MODEL), lambda e, f: (e, f, 0)),
            pl.BlockSpec((1, 1, D_MODEL), lambda e, f: (e, 0, 0)),
        ],
        out_specs=pl.BlockSpec((1, CAPACITY, D_MODEL), lambda e, f: (e, 0, 0)),
        out_shape=jax.ShapeDtypeStruct((E, CAPACITY, D_MODEL), jnp.float32),
        scratch_shapes=[pltpu.VMEM((CAPACITY, D_MODEL), jnp.float32)],
    )(bufs_bf, w1b, b1, w2b, b2)


def _combine_body(slotc_hbm, gate_hbm, y_hbm, out_hbm,
                  slot_v, gate_v, rows_a, rows_b, out_a, out_b,
                  sem_ga, sem_gb, sem_wa, sem_wb):
    cid = lax.axis_index("c")
    sid = lax.axis_index("s")
    wid = sid * 2 + cid
    base_e = wid * TPT * TOP_K

    pltpu.sync_copy(slotc_hbm.at[pl.ds(base_e, TPT * TOP_K)], slot_v)
    pltpu.sync_copy(gate_hbm.at[pl.ds(base_e, TPT * TOP_K)],
                    gate_v.at[pl.ds(0, TPT * TOP_K)])

    rows = [rows_a, rows_b]
    outs = [out_a, out_b]
    sem_g = [sem_ga, sem_gb]
    sem_w = [sem_wa, sem_wb]
    NCH = 4

    gcp = [None] * NCH
    wcp = [None] * NCH
    gcp[0] = pltpu.async_copy(
        y_hbm.at[slot_v.at[pl.ds(0, 32)]], rows[0], sem_g[0])
    for k in range(NCH):
        p = k & 1
        gcp[k].wait()
        if k + 1 < NCH:
            gcp[k + 1] = pltpu.async_copy(
                y_hbm.at[slot_v.at[pl.ds((k + 1) * 32, 32)]],
                rows[1 - p], sem_g[1 - p])
        if k >= 2:
            wcp[k - 2].wait()
        rv = rows[p]
        ov = outs[p]

        def tok_i(i, _):
            gv = gate_v[pl.ds(k * 32 + 2 * i, 16)]
            g0 = gv[0]
            g1 = gv[1]

            def col_j(j, _):
                r0 = rv[2 * i, pl.ds(j * 16, 16)]
                r1 = rv[2 * i + 1, pl.ds(j * 16, 16)]
                ov[i, pl.ds(j * 16, 16)] = g0 * r0 + g1 * r1
                return 0
            lax.fori_loop(0, D_MODEL // 16, col_j, 0)
            return 0
        lax.fori_loop(0, 16, tok_i, 0)
        wcp[k] = pltpu.async_copy(
            ov, out_hbm.at[pl.ds(wid * TPT + k * 16, 16)], sem_w[p])
    wcp[NCH - 2].wait()
    wcp[NCH - 1].wait()


def _combine(slotc_flat, gate_flat, y_flat):
    mesh = plsc.VectorSubcoreMesh(core_axis_name="c", subcore_axis_name="s", num_cores=2, num_subcores=16)
    return pl.kernel(
        _combine_body,
        out_type=jax.ShapeDtypeStruct((T, D_MODEL), jnp.float32),
        mesh=mesh,
        compiler_params=pltpu.CompilerParams(needs_layout_passes=False),
        scratch_types=[
            pltpu.VMEM((TPT * TOP_K,), jnp.int32),
            pltpu.VMEM((TPT * TOP_K + 32,), jnp.float32),
            pltpu.VMEM((32, D_MODEL), jnp.float32),
            pltpu.VMEM((32, D_MODEL), jnp.float32),
            pltpu.VMEM((16, D_MODEL), jnp.float32),
            pltpu.VMEM((16, D_MODEL), jnp.float32),
            pltpu.SemaphoreType.DMA,
            pltpu.SemaphoreType.DMA,
            pltpu.SemaphoreType.DMA,
            pltpu.SemaphoreType.DMA,
        ],
    )(slotc_flat, gate_flat, y_flat)


def kernel(x, Wg, w1, b1, w2, b2):
    slotd, slotc, gate = _router(x, Wg)
    xi = lax.bitcast_convert_type(
        x.astype(jnp.bfloat16).reshape(T, D_MODEL // 2, 2), jnp.int32)
    buffers = _dispatch(slotd.reshape(-1), xi)
    bufs_bf = lax.bitcast_convert_type(
        buffers.reshape(NSLOT, D_MODEL // 2, 1), jnp.bfloat16).reshape(
        E, CAPACITY, D_MODEL)
    y = _ffn(bufs_bf, w1.astype(jnp.bfloat16), b1[:, None, :],
             w2.astype(jnp.bfloat16), b2[:, None, :])
    out = _combine(slotc.reshape(-1), gate.reshape(-1),
                   y.reshape(NSLOT, D_MODEL))
    return out


# speedup vs baseline: 1.3171x; 1.3171x over previous
import functools

import jax
import jax.numpy as jnp
from jax import lax
from jax.experimental import pallas as pl
from jax.experimental.pallas import tpu as pltpu
from jax.experimental.pallas import tpu_sc as plsc

D_MODEL = 1024
D_FF = 4096
E = 8
TOP_K = 2
T = 2048
CAPACITY = 640
NSLOT = E * CAPACITY
TK = T * TOP_K

NW = 32
SPT = NSLOT // NW
TPT = T // NW

RBLK = 128
NRB = T // RBLK

FBLK = 2048
NFB = D_FF // FBLK


def _router_body(x_ref, wg_ref, slotd_ref, slotc_ref, gate_ref, carry_ref):
    i = pl.program_id(0)

    @pl.when(i == 0)
    def _():
        carry_ref[...] = jnp.zeros_like(carry_ref)

    xb = x_ref[...]
    wg = wg_ref[...]
    logits = jnp.dot(xb, wg, preferred_element_type=jnp.float32)
    probs = jax.nn.softmax(logits, axis=-1)

    ids = lax.broadcasted_iota(jnp.int32, (RBLK, E), 1)
    m0 = jnp.max(probs, axis=-1, keepdims=True)
    am0 = jnp.min(jnp.where(probs == m0, ids, E), axis=-1, keepdims=True)
    probs2 = jnp.where(ids == am0, -1.0, probs)
    m1 = jnp.max(probs2, axis=-1, keepdims=True)
    am1 = jnp.min(jnp.where(probs2 == m1, ids, E), axis=-1, keepdims=True)
    gsum = m0 + m1 + 1e-9
    g0 = m0 / gsum
    g1 = m1 / gsum

    oh0 = (ids == am0).astype(jnp.float32)
    oh1 = (ids == am1).astype(jnp.float32)
    cnt = oh0 + oh1

    r = lax.broadcasted_iota(jnp.int32, (RBLK, RBLK), 0)
    c = lax.broadcasted_iota(jnp.int32, (RBLK, RBLK), 1)
    tri = (r > c).astype(jnp.float32)
    base = jnp.dot(tri, cnt, preferred_element_type=jnp.float32)
    base = base + carry_ref[...]
    carry_ref[...] = carry_ref[...] + jnp.sum(cnt, axis=0, keepdims=True)

    pos0 = jnp.sum(base * oh0, axis=-1, keepdims=True).astype(jnp.int32)
    pos1 = jnp.sum(base * oh1, axis=-1, keepdims=True).astype(jnp.int32)
    keep0 = pos0 < CAPACITY
    keep1 = pos1 < CAPACITY
    slot0 = am0 * CAPACITY + pos0
    slot1 = am1 * CAPACITY + pos1

    slotd_ref[...] = jnp.concatenate(
        [jnp.where(keep0, slot0, -1), jnp.where(keep1, slot1, -1)], axis=1)
    slotc_ref[...] = jnp.concatenate(
        [jnp.where(keep0, slot0, 0), jnp.where(keep1, slot1, 0)], axis=1)
    gate_ref[...] = jnp.concatenate(
        [g0 * keep0.astype(jnp.float32), g1 * keep1.astype(jnp.float32)], axis=1)


def _router(x, Wg):
    return pl.pallas_call(
        _router_body,
        grid=(NRB,),
        in_specs=[
            pl.BlockSpec((RBLK, D_MODEL), lambda i: (i, 0)),
            pl.BlockSpec((D_MODEL, E), lambda i: (0, 0)),
        ],
        out_specs=[
            pl.BlockSpec((RBLK, TOP_K), lambda i: (i, 0)),
            pl.BlockSpec((RBLK, TOP_K), lambda i: (i, 0)),
            pl.BlockSpec((RBLK, TOP_K), lambda i: (i, 0)),
        ],
        out_shape=[
            jax.ShapeDtypeStruct((T, TOP_K), jnp.int32),
            jax.ShapeDtypeStruct((T, TOP_K), jnp.int32),
            jax.ShapeDtypeStruct((T, TOP_K), jnp.float32),
        ],
        scratch_shapes=[pltpu.VMEM((1, E), jnp.float32)],
    )(x, Wg)


EPS = TK // 16


def _dispatch_body(slotd_hbm, x_hbm, out_hbm,
                   slot_v, idx_v, val_v, zero_v, src_v,
                   rows_a, rows_b, src_sp,
                   sem_g0, sem_g1, sem_w0, sem_w1, sem_s):
    cid = lax.axis_index("c")
    sid = lax.axis_index("s")
    wid = sid * 2 + cid
    lo = wid * SPT

    def z_i(i, _):
        zero_v[pl.ds(i * 16, 16)] = jnp.zeros((16,), jnp.int32)
        return 0
    lax.fori_loop(0, (NSLOT // 16) // 16, z_i, 0)
    pltpu.sync_copy(zero_v, src_sp.at[pl.ds(sid * (NSLOT // 16), NSLOT // 16)])

    cp_s = pltpu.async_copy(slotd_hbm.at[pl.ds(sid * EPS, EPS)], slot_v, sem_s)

    plsc.subcore_barrier()

    cp_s.wait()

    def chunk_j(j):
        def e_i(i, _):
            base = j * 128 + i * 16
            sv = slot_v[pl.ds(base, 16)]
            m = sv >= 0
            tvec = lax.shift_right_logical(
                sid * EPS + base + lax.iota(jnp.int32, 16), 1)
            idx_v[j, pl.ds(i * 16, 16)] = jnp.where(m, sv, 0)
            val_v[j, pl.ds(i * 16, 16)] = jnp.where(m, tvec, 0)
            return 0
        lax.fori_loop(0, 8, e_i, 0)

    for j in range(2):
        chunk_j(j)
        pltpu.sync_copy(val_v.at[j], src_sp.at[idx_v.at[j]], add=True)

    plsc.subcore_barrier()

    pltpu.sync_copy(src_sp.at[pl.ds(lo, SPT)], src_v)

    rows = [rows_a, rows_b]
    sem_g = [sem_g0, sem_g1]
    sem_w = [sem_w0, sem_w1]
    RCH = 40
    NCH = SPT // RCH
    gcp = [None] * NCH
    wcp = [None] * NCH
    gcp[0] = pltpu.async_copy(
        x_hbm.at[src_v.at[pl.ds(0, RCH)]], rows[0], sem_g[0])
    for k in range(NCH):
        p = k & 1
        gcp[k].wait()
        if k + 1 < NCH:
            if k >= 1:
                wcp[k - 1].wait()
            gcp[k + 1] = pltpu.async_copy(
                x_hbm.at[src_v.at[pl.ds((k + 1) * RCH, RCH)]],
                rows[1 - p], sem_g[1 - p])
        wcp[k] = pltpu.async_copy(
            rows[p], out_hbm.at[pl.ds(lo + k * RCH, RCH)], sem_w[p])
    wcp[NCH - 2].wait()
    wcp[NCH - 1].wait()


def _dispatch(slotd_flat, xbf):
    mesh = plsc.VectorSubcoreMesh(core_axis_name="c", subcore_axis_name="s", num_cores=2, num_subcores=16)
    return pl.kernel(
        _dispatch_body,
        out_type=jax.ShapeDtypeStruct((NSLOT, D_MODEL), jnp.float32),
        mesh=mesh,
        compiler_params=pltpu.CompilerParams(needs_layout_passes=False),
        scratch_types=[
            pltpu.VMEM((EPS,), jnp.int32),
            pltpu.VMEM((2, 128), jnp.int32),
            pltpu.VMEM((2, 128), jnp.int32),
            pltpu.VMEM((NSLOT // 16,), jnp.int32),
            pltpu.VMEM((SPT,), jnp.int32),
            pltpu.VMEM((40, D_MODEL), jnp.float32),
            pltpu.VMEM((40, D_MODEL), jnp.float32),
            pltpu.VMEM_SHARED((NSLOT,), jnp.int32),
            pltpu.SemaphoreType.DMA,
            pltpu.SemaphoreType.DMA,
            pltpu.SemaphoreType.DMA,
            pltpu.SemaphoreType.DMA,
            pltpu.SemaphoreType.DMA,
        ],
    )(slotd_flat, xbf)


def _ffn_body(a_ref, w1_ref, b1_ref, w2_ref, b2_ref, y_ref, acc_ref):
    f = pl.program_id(1)
    a = a_ref[0].astype(jnp.bfloat16)
    h = jnp.dot(a, w1_ref[0], preferred_element_type=jnp.float32)
    h = jnp.maximum(h + b1_ref[0], 0.0)
    hb = h.astype(jnp.bfloat16)
    part = jnp.dot(hb, w2_ref[0], preferred_element_type=jnp.float32)

    @pl.when(f == 0)
    def _():
        acc_ref[...] = part

    @pl.when(f != 0)
    def _():
        acc_ref[...] = acc_ref[...] + part

    @pl.when(f == NFB - 1)
    def _():
        y_ref[0] = acc_ref[...] + b2_ref[0]


def _ffn(bufs_bf, w1b, b1, w2b, b2):
    return pl.pallas_call(
        _ffn_body,
        grid=(E, NFB),
        in_specs=[
            pl.BlockSpec((1, CAPACITY, D_MODEL), lambda e, f: (e, 0, 0)),
            pl.BlockSpec((1, D_MODEL, FBLK), lambda e, f: (e, 0, f)),
            pl.BlockSpec((1, 1, FBLK), lambda e, f: (e, 0, f)),
            pl.BlockSpec((1, FBLK, D_MODEL), lambda e, f: (e, f, 0)),
            pl.BlockSpec((1, 1, D_MODEL), lambda e, f: (e, 0, 0)),
        ],
        out_specs=pl.BlockSpec((1, CAPACITY, D_MODEL), lambda e, f: (e, 0, 0)),
        out_shape=jax.ShapeDtypeStruct((E, CAPACITY, D_MODEL), jnp.float32),
        scratch_shapes=[pltpu.VMEM((CAPACITY, D_MODEL), jnp.float32)],
    )(bufs_bf, w1b, b1, w2b, b2)


def _combine_body(slotc_hbm, gate_hbm, y_hbm, out_hbm,
                  slot_v, gate_v, rows_a, rows_b, out_a, out_b,
                  sem_ga, sem_gb, sem_wa, sem_wb):
    cid = lax.axis_index("c")
    sid = lax.axis_index("s")
    wid = sid * 2 + cid
    base_e = wid * TPT * TOP_K

    pltpu.sync_copy(slotc_hbm.at[pl.ds(base_e, TPT * TOP_K)], slot_v)
    pltpu.sync_copy(gate_hbm.at[pl.ds(base_e, TPT * TOP_K)],
                    gate_v.at[pl.ds(0, TPT * TOP_K)])

    rows = [rows_a, rows_b]
    outs = [out_a, out_b]
    sem_g = [sem_ga, sem_gb]
    sem_w = [sem_wa, sem_wb]
    NCH = 4

    gcp = [None] * NCH
    wcp = [None] * NCH
    gcp[0] = pltpu.async_copy(
        y_hbm.at[slot_v.at[pl.ds(0, 32)]], rows[0], sem_g[0])
    for k in range(NCH):
        p = k & 1
        gcp[k].wait()
        if k + 1 < NCH:
            gcp[k + 1] = pltpu.async_copy(
                y_hbm.at[slot_v.at[pl.ds((k + 1) * 32, 32)]],
                rows[1 - p], sem_g[1 - p])
        if k >= 2:
            wcp[k - 2].wait()
        rv = rows[p]
        ov = outs[p]

        def tok_i(i, _):
            gv = gate_v[pl.ds(k * 32 + 2 * i, 16)]
            g0 = gv[0]
            g1 = gv[1]

            def col_j(j, _):
                r0 = rv[2 * i, pl.ds(j * 16, 16)]
                r1 = rv[2 * i + 1, pl.ds(j * 16, 16)]
                ov[i, pl.ds(j * 16, 16)] = g0 * r0 + g1 * r1
                return 0
            lax.fori_loop(0, D_MODEL // 16, col_j, 0)
            return 0
        lax.fori_loop(0, 16, tok_i, 0)
        wcp[k] = pltpu.async_copy(
            ov, out_hbm.at[pl.ds(wid * TPT + k * 16, 16)], sem_w[p])
    wcp[NCH - 2].wait()
    wcp[NCH - 1].wait()


def _combine(slotc_flat, gate_flat, y_flat):
    mesh = plsc.VectorSubcoreMesh(core_axis_name="c", subcore_axis_name="s", num_cores=2, num_subcores=16)
    return pl.kernel(
        _combine_body,
        out_type=jax.ShapeDtypeStruct((T, D_MODEL), jnp.float32),
        mesh=mesh,
        compiler_params=pltpu.CompilerParams(needs_layout_passes=False),
        scratch_types=[
            pltpu.VMEM((TPT * TOP_K,), jnp.int32),
            pltpu.VMEM((TPT * TOP_K + 32,), jnp.float32),
            pltpu.VMEM((32, D_MODEL), jnp.float32),
            pltpu.VMEM((32, D_MODEL), jnp.float32),
            pltpu.VMEM((16, D_MODEL), jnp.float32),
            pltpu.VMEM((16, D_MODEL), jnp.float32),
            pltpu.SemaphoreType.DMA,
            pltpu.SemaphoreType.DMA,
            pltpu.SemaphoreType.DMA,
            pltpu.SemaphoreType.DMA,
        ],
    )(slotc_flat, gate_flat, y_flat)


def kernel(x, Wg, w1, b1, w2, b2):
    slotd, slotc, gate = _router(x, Wg)
    buffers = _dispatch(slotd.reshape(-1), x)
    bufs = buffers.reshape(E, CAPACITY, D_MODEL)
    y = _ffn(bufs, w1.astype(jnp.bfloat16), b1[:, None, :],
             w2.astype(jnp.bfloat16), b2[:, None, :])
    out = _combine(slotc.reshape(-1), gate.reshape(-1),
                   y.reshape(NSLOT, D_MODEL))
    return out


# speedup vs baseline: 1.8997x; 1.4424x over previous
import functools

import jax
import jax.numpy as jnp
from jax import lax
from jax.experimental import pallas as pl
from jax.experimental.pallas import tpu as pltpu
from jax.experimental.pallas import tpu_sc as plsc

D_MODEL = 1024
D_FF = 4096
E = 8
TOP_K = 2
T = 2048
CAPACITY = 640
NSLOT = E * CAPACITY
TK = T * TOP_K

NW = 32
SPT = NSLOT // NW
TPT = T // NW

RBLK = 128
NRB = T // RBLK

FBLK = 2048
NFB = D_FF // FBLK


def _router_body(x_ref, wg_ref, slotd_ref, slotc_ref, gate_ref, carry_ref):
    i = pl.program_id(0)

    @pl.when(i == 0)
    def _():
        carry_ref[...] = jnp.zeros_like(carry_ref)

    xb = x_ref[...]
    wg = wg_ref[...]
    logits = jnp.dot(xb, wg, preferred_element_type=jnp.float32)
    probs = jax.nn.softmax(logits, axis=-1)

    ids = lax.broadcasted_iota(jnp.int32, (RBLK, E), 1)
    m0 = jnp.max(probs, axis=-1, keepdims=True)
    am0 = jnp.min(jnp.where(probs == m0, ids, E), axis=-1, keepdims=True)
    probs2 = jnp.where(ids == am0, -1.0, probs)
    m1 = jnp.max(probs2, axis=-1, keepdims=True)
    am1 = jnp.min(jnp.where(probs2 == m1, ids, E), axis=-1, keepdims=True)
    gsum = m0 + m1 + 1e-9
    g0 = m0 / gsum
    g1 = m1 / gsum

    oh0 = (ids == am0).astype(jnp.float32)
    oh1 = (ids == am1).astype(jnp.float32)
    cnt = oh0 + oh1

    r = lax.broadcasted_iota(jnp.int32, (RBLK, RBLK), 0)
    c = lax.broadcasted_iota(jnp.int32, (RBLK, RBLK), 1)
    tri = (r > c).astype(jnp.float32)
    base = jnp.dot(tri, cnt, preferred_element_type=jnp.float32)
    base = base + carry_ref[...]
    carry_ref[...] = carry_ref[...] + jnp.sum(cnt, axis=0, keepdims=True)

    pos0 = jnp.sum(base * oh0, axis=-1, keepdims=True).astype(jnp.int32)
    pos1 = jnp.sum(base * oh1, axis=-1, keepdims=True).astype(jnp.int32)
    keep0 = pos0 < CAPACITY
    keep1 = pos1 < CAPACITY
    slot0 = am0 * CAPACITY + pos0
    slot1 = am1 * CAPACITY + pos1

    slotd_ref[...] = jnp.concatenate(
        [jnp.where(keep0, slot0, -1), jnp.where(keep1, slot1, -1)], axis=1)
    slotc_ref[...] = jnp.concatenate(
        [jnp.where(keep0, slot0, 0), jnp.where(keep1, slot1, 0)], axis=1)
    gate_ref[...] = jnp.concatenate(
        [g0 * keep0.astype(jnp.float32), g1 * keep1.astype(jnp.float32)], axis=1)


def _router(x, Wg):
    return pl.pallas_call(
        _router_body,
        grid=(NRB,),
        in_specs=[
            pl.BlockSpec((RBLK, D_MODEL), lambda i: (i, 0)),
            pl.BlockSpec((D_MODEL, E), lambda i: (0, 0)),
        ],
        out_specs=[
            pl.BlockSpec((RBLK, TOP_K), lambda i: (i, 0)),
            pl.BlockSpec((RBLK, TOP_K), lambda i: (i, 0)),
            pl.BlockSpec((RBLK, TOP_K), lambda i: (i, 0)),
        ],
        out_shape=[
            jax.ShapeDtypeStruct((T, TOP_K), jnp.int32),
            jax.ShapeDtypeStruct((T, TOP_K), jnp.int32),
            jax.ShapeDtypeStruct((T, TOP_K), jnp.float32),
        ],
        scratch_shapes=[pltpu.VMEM((1, E), jnp.float32)],
    )(x, Wg)


EPS = TK // 16


def _dispatch_body(slotd_hbm, x_hbm, out_hbm,
                   slot_v, idx_v, val_v, zero_v, src_v,
                   rows_a, rows_b, src_sp,
                   sem_g0, sem_g1, sem_w0, sem_w1, sem_s):
    cid = lax.axis_index("c")
    sid = lax.axis_index("s")
    wid = sid * 2 + cid
    lo = wid * SPT

    def z_i(i, _):
        zero_v[pl.ds(i * 16, 16)] = jnp.zeros((16,), jnp.int32)
        return 0
    lax.fori_loop(0, (NSLOT // 16) // 16, z_i, 0)
    pltpu.sync_copy(zero_v, src_sp.at[pl.ds(sid * (NSLOT // 16), NSLOT // 16)])

    cp_s = pltpu.async_copy(slotd_hbm.at[pl.ds(sid * EPS, EPS)], slot_v, sem_s)

    plsc.subcore_barrier()

    cp_s.wait()

    def chunk_j(j):
        def e_i(i, _):
            base = j * 128 + i * 16
            sv = slot_v[pl.ds(base, 16)]
            m = sv >= 0
            tvec = lax.shift_right_logical(
                sid * EPS + base + lax.iota(jnp.int32, 16), 1)
            idx_v[j, pl.ds(i * 16, 16)] = jnp.where(m, sv, 0)
            val_v[j, pl.ds(i * 16, 16)] = jnp.where(m, tvec, 0)
            return 0
        lax.fori_loop(0, 8, e_i, 0)

    for j in range(2):
        chunk_j(j)
        pltpu.sync_copy(val_v.at[j], src_sp.at[idx_v.at[j]], add=True)

    plsc.subcore_barrier()

    pltpu.sync_copy(src_sp.at[pl.ds(lo, SPT)], src_v)

    rows = [rows_a, rows_b]
    sem_g = [sem_g0, sem_g1]
    sem_w = [sem_w0, sem_w1]
    RCH = 40
    NCH = SPT // RCH
    gcp = [None] * NCH
    wcp = [None] * NCH
    gcp[0] = pltpu.async_copy(
        x_hbm.at[src_v.at[pl.ds(0, RCH)]], rows[0], sem_g[0])
    for k in range(NCH):
        p = k & 1
        gcp[k].wait()
        if k + 1 < NCH:
            if k >= 1:
                wcp[k - 1].wait()
            gcp[k + 1] = pltpu.async_copy(
                x_hbm.at[src_v.at[pl.ds((k + 1) * RCH, RCH)]],
                rows[1 - p], sem_g[1 - p])
        wcp[k] = pltpu.async_copy(
            rows[p], out_hbm.at[pl.ds(lo + k * RCH, RCH)], sem_w[p])
    wcp[NCH - 2].wait()
    wcp[NCH - 1].wait()


def _dispatch(slotd_flat, xbf):
    mesh = plsc.VectorSubcoreMesh(core_axis_name="c", subcore_axis_name="s", num_cores=2, num_subcores=16)
    return pl.kernel(
        _dispatch_body,
        out_type=jax.ShapeDtypeStruct((NSLOT, D_MODEL), jnp.float32),
        mesh=mesh,
        compiler_params=pltpu.CompilerParams(needs_layout_passes=False),
        scratch_types=[
            pltpu.VMEM((EPS,), jnp.int32),
            pltpu.VMEM((2, 128), jnp.int32),
            pltpu.VMEM((2, 128), jnp.int32),
            pltpu.VMEM((NSLOT // 16,), jnp.int32),
            pltpu.VMEM((SPT,), jnp.int32),
            pltpu.VMEM((40, D_MODEL), jnp.float32),
            pltpu.VMEM((40, D_MODEL), jnp.float32),
            pltpu.VMEM_SHARED((NSLOT,), jnp.int32),
            pltpu.SemaphoreType.DMA,
            pltpu.SemaphoreType.DMA,
            pltpu.SemaphoreType.DMA,
            pltpu.SemaphoreType.DMA,
            pltpu.SemaphoreType.DMA,
        ],
    )(slotd_flat, xbf)


def _ffn_body(a_ref, w1_ref, b1_ref, w2_ref, b2_ref, y_ref, acc_ref):
    f = pl.program_id(1)
    a = a_ref[0]
    h = jnp.dot(a, w1_ref[0], preferred_element_type=jnp.float32)
    h = jnp.maximum(h + b1_ref[0], 0.0)
    part = jnp.dot(h, w2_ref[0], preferred_element_type=jnp.float32)

    @pl.when(f == 0)
    def _():
        acc_ref[...] = part

    @pl.when(f != 0)
    def _():
        acc_ref[...] = acc_ref[...] + part

    @pl.when(f == NFB - 1)
    def _():
        y_ref[0] = acc_ref[...] + b2_ref[0]


def _ffn(bufs_bf, w1b, b1, w2b, b2):
    return pl.pallas_call(
        _ffn_body,
        grid=(E, NFB),
        in_specs=[
            pl.BlockSpec((1, CAPACITY, D_MODEL), lambda e, f: (e, 0, 0)),
            pl.BlockSpec((1, D_MODEL, FBLK), lambda e, f: (e, 0, f)),
            pl.BlockSpec((1, 1, FBLK), lambda e, f: (e, 0, f)),
            pl.BlockSpec((1, FBLK, D_MODEL), lambda e, f: (e, f, 0)),
            pl.BlockSpec((1, 1, D_MODEL), lambda e, f: (e, 0, 0)),
        ],
        out_specs=pl.BlockSpec((1, CAPACITY, D_MODEL), lambda e, f: (e, 0, 0)),
        out_shape=jax.ShapeDtypeStruct((E, CAPACITY, D_MODEL), jnp.float32),
        scratch_shapes=[pltpu.VMEM((CAPACITY, D_MODEL), jnp.float32)],
    )(bufs_bf, w1b, b1, w2b, b2)


def _combine_body(slotc_hbm, gate_hbm, y_hbm, out_hbm,
                  slot_v, gate_v, rows_a, rows_b, out_a, out_b,
                  sem_ga, sem_gb, sem_wa, sem_wb):
    cid = lax.axis_index("c")
    sid = lax.axis_index("s")
    wid = sid * 2 + cid
    base_e = wid * TPT * TOP_K

    pltpu.sync_copy(slotc_hbm.at[pl.ds(base_e, TPT * TOP_K)], slot_v)
    pltpu.sync_copy(gate_hbm.at[pl.ds(base_e, TPT * TOP_K)],
                    gate_v.at[pl.ds(0, TPT * TOP_K)])

    rows = [rows_a, rows_b]
    outs = [out_a, out_b]
    sem_g = [sem_ga, sem_gb]
    sem_w = [sem_wa, sem_wb]
    NCH = 4

    gcp = [None] * NCH
    wcp = [None] * NCH
    gcp[0] = pltpu.async_copy(
        y_hbm.at[slot_v.at[pl.ds(0, 32)]], rows[0], sem_g[0])
    for k in range(NCH):
        p = k & 1
        gcp[k].wait()
        if k + 1 < NCH:
            gcp[k + 1] = pltpu.async_copy(
                y_hbm.at[slot_v.at[pl.ds((k + 1) * 32, 32)]],
                rows[1 - p], sem_g[1 - p])
        if k >= 2:
            wcp[k - 2].wait()
        rv = rows[p]
        ov = outs[p]

        def tok_i(i, _):
            gv = gate_v[pl.ds(k * 32 + 2 * i, 16)]
            g0 = gv[0]
            g1 = gv[1]

            def col_j(j, _):
                r0 = rv[2 * i, pl.ds(j * 16, 16)]
                r1 = rv[2 * i + 1, pl.ds(j * 16, 16)]
                ov[i, pl.ds(j * 16, 16)] = g0 * r0 + g1 * r1
                return 0
            lax.fori_loop(0, D_MODEL // 16, col_j, 0)
            return 0
        lax.fori_loop(0, 16, tok_i, 0)
        wcp[k] = pltpu.async_copy(
            ov, out_hbm.at[pl.ds(wid * TPT + k * 16, 16)], sem_w[p])
    wcp[NCH - 2].wait()
    wcp[NCH - 1].wait()


def _combine(slotc_flat, gate_flat, y_flat):
    mesh = plsc.VectorSubcoreMesh(core_axis_name="c", subcore_axis_name="s", num_cores=2, num_subcores=16)
    return pl.kernel(
        _combine_body,
        out_type=jax.ShapeDtypeStruct((T, D_MODEL), jnp.float32),
        mesh=mesh,
        compiler_params=pltpu.CompilerParams(needs_layout_passes=False),
        scratch_types=[
            pltpu.VMEM((TPT * TOP_K,), jnp.int32),
            pltpu.VMEM((TPT * TOP_K + 32,), jnp.float32),
            pltpu.VMEM((32, D_MODEL), jnp.float32),
            pltpu.VMEM((32, D_MODEL), jnp.float32),
            pltpu.VMEM((16, D_MODEL), jnp.float32),
            pltpu.VMEM((16, D_MODEL), jnp.float32),
            pltpu.SemaphoreType.DMA,
            pltpu.SemaphoreType.DMA,
            pltpu.SemaphoreType.DMA,
            pltpu.SemaphoreType.DMA,
        ],
    )(slotc_flat, gate_flat, y_flat)


def kernel(x, Wg, w1, b1, w2, b2):
    slotd, slotc, gate = _router(x, Wg)
    buffers = _dispatch(slotd.reshape(-1), x)
    bufs = buffers.reshape(E, CAPACITY, D_MODEL)
    y = _ffn(bufs, w1, b1[:, None, :],
             w2, b2[:, None, :])
    out = _combine(slotc.reshape(-1), gate.reshape(-1),
                   y.reshape(NSLOT, D_MODEL))
    return out


# speedup vs baseline: 1.9445x; 1.0236x over previous
import functools

import jax
import jax.numpy as jnp
from jax import lax
from jax.experimental import pallas as pl
from jax.experimental.pallas import tpu as pltpu
from jax.experimental.pallas import tpu_sc as plsc

D_MODEL = 1024
D_FF = 4096
E = 8
TOP_K = 2
T = 2048
CAPACITY = 640
NSLOT = E * CAPACITY
TK = T * TOP_K

NW = 32
SPT = NSLOT // NW
TPT = T // NW

RBLK = 256
NRB = T // RBLK

FBLK = 2048
NFB = D_FF // FBLK


def _router_body(x_ref, wg_ref, slotd_ref, slotc_ref, gate_ref, carry_ref):
    i = pl.program_id(0)

    @pl.when(i == 0)
    def _():
        carry_ref[...] = jnp.zeros_like(carry_ref)

    xb = x_ref[...]
    wg = wg_ref[...]
    logits = jnp.dot(xb, wg, preferred_element_type=jnp.float32)
    probs = jax.nn.softmax(logits, axis=-1)

    ids = lax.broadcasted_iota(jnp.int32, (RBLK, E), 1)
    m0 = jnp.max(probs, axis=-1, keepdims=True)
    am0 = jnp.min(jnp.where(probs == m0, ids, E), axis=-1, keepdims=True)
    probs2 = jnp.where(ids == am0, -1.0, probs)
    m1 = jnp.max(probs2, axis=-1, keepdims=True)
    am1 = jnp.min(jnp.where(probs2 == m1, ids, E), axis=-1, keepdims=True)
    gsum = m0 + m1 + 1e-9
    g0 = m0 / gsum
    g1 = m1 / gsum

    oh0 = (ids == am0).astype(jnp.float32)
    oh1 = (ids == am1).astype(jnp.float32)
    cnt = oh0 + oh1

    r = lax.broadcasted_iota(jnp.int32, (RBLK, RBLK), 0)
    c = lax.broadcasted_iota(jnp.int32, (RBLK, RBLK), 1)
    tri = (r > c).astype(jnp.float32)
    base = jnp.dot(tri, cnt, preferred_element_type=jnp.float32)
    base = base + carry_ref[...]
    carry_ref[...] = carry_ref[...] + jnp.sum(cnt, axis=0, keepdims=True)

    pos0 = jnp.sum(base * oh0, axis=-1, keepdims=True).astype(jnp.int32)
    pos1 = jnp.sum(base * oh1, axis=-1, keepdims=True).astype(jnp.int32)
    keep0 = pos0 < CAPACITY
    keep1 = pos1 < CAPACITY
    slot0 = am0 * CAPACITY + pos0
    slot1 = am1 * CAPACITY + pos1

    slotd_ref[...] = jnp.concatenate(
        [jnp.where(keep0, slot0, -1), jnp.where(keep1, slot1, -1)], axis=1)
    slotc_ref[...] = jnp.concatenate(
        [jnp.where(keep0, slot0, 0), jnp.where(keep1, slot1, 0)], axis=1)
    gate_ref[...] = jnp.concatenate(
        [g0 * keep0.astype(jnp.float32), g1 * keep1.astype(jnp.float32)], axis=1)


def _router(x, Wg):
    return pl.pallas_call(
        _router_body,
        grid=(NRB,),
        in_specs=[
            pl.BlockSpec((RBLK, D_MODEL), lambda i: (i, 0)),
            pl.BlockSpec((D_MODEL, E), lambda i: (0, 0)),
        ],
        out_specs=[
            pl.BlockSpec((RBLK, TOP_K), lambda i: (i, 0)),
            pl.BlockSpec((RBLK, TOP_K), lambda i: (i, 0)),
            pl.BlockSpec((RBLK, TOP_K), lambda i: (i, 0)),
        ],
        out_shape=[
            jax.ShapeDtypeStruct((T, TOP_K), jnp.int32),
            jax.ShapeDtypeStruct((T, TOP_K), jnp.int32),
            jax.ShapeDtypeStruct((T, TOP_K), jnp.float32),
        ],
        scratch_shapes=[pltpu.VMEM((1, E), jnp.float32)],
    )(x, Wg)


EPS = TK // 16


def _dispatch_body(slotd_hbm, x_hbm, out_hbm,
                   slot_v, idx_v, val_v, zero_v, src_v,
                   rows_a, rows_b, src_sp,
                   sem_g0, sem_g1, sem_w0, sem_w1, sem_s):
    cid = lax.axis_index("c")
    sid = lax.axis_index("s")
    wid = sid * 2 + cid
    lo = wid * SPT

    def z_i(i, _):
        zero_v[pl.ds(i * 16, 16)] = jnp.zeros((16,), jnp.int32)
        return 0
    lax.fori_loop(0, (NSLOT // 16) // 16, z_i, 0)
    pltpu.sync_copy(zero_v, src_sp.at[pl.ds(sid * (NSLOT // 16), NSLOT // 16)])

    cp_s = pltpu.async_copy(slotd_hbm.at[pl.ds(sid * EPS, EPS)], slot_v, sem_s)

    plsc.subcore_barrier()

    cp_s.wait()

    def chunk_j(j):
        def e_i(i, _):
            base = j * 128 + i * 16
            sv = slot_v[pl.ds(base, 16)]
            m = sv >= 0
            tvec = lax.shift_right_logical(
                sid * EPS + base + lax.iota(jnp.int32, 16), 1)
            idx_v[j, pl.ds(i * 16, 16)] = jnp.where(m, sv, 0)
            val_v[j, pl.ds(i * 16, 16)] = jnp.where(m, tvec, 0)
            return 0
        lax.fori_loop(0, 8, e_i, 0)

    for j in range(2):
        chunk_j(j)
        pltpu.sync_copy(val_v.at[j], src_sp.at[idx_v.at[j]], add=True)

    plsc.subcore_barrier()

    pltpu.sync_copy(src_sp.at[pl.ds(lo, SPT)], src_v)

    rows = [rows_a, rows_b]
    sem_g = [sem_g0, sem_g1]
    sem_w = [sem_w0, sem_w1]
    RCH = 40
    NCH = SPT // RCH
    gcp = [None] * NCH
    wcp = [None] * NCH
    gcp[0] = pltpu.async_copy(
        x_hbm.at[src_v.at[pl.ds(0, RCH)]], rows[0], sem_g[0])
    for k in range(NCH):
        p = k & 1
        gcp[k].wait()
        if k + 1 < NCH:
            if k >= 1:
                wcp[k - 1].wait()
            gcp[k + 1] = pltpu.async_copy(
                x_hbm.at[src_v.at[pl.ds((k + 1) * RCH, RCH)]],
                rows[1 - p], sem_g[1 - p])
        wcp[k] = pltpu.async_copy(
            rows[p], out_hbm.at[pl.ds(lo + k * RCH, RCH)], sem_w[p])
    wcp[NCH - 2].wait()
    wcp[NCH - 1].wait()


def _dispatch(slotd_flat, xbf):
    mesh = plsc.VectorSubcoreMesh(core_axis_name="c", subcore_axis_name="s", num_cores=2, num_subcores=16)
    return pl.kernel(
        _dispatch_body,
        out_type=jax.ShapeDtypeStruct((NSLOT, D_MODEL), jnp.float32),
        mesh=mesh,
        compiler_params=pltpu.CompilerParams(needs_layout_passes=False),
        scratch_types=[
            pltpu.VMEM((EPS,), jnp.int32),
            pltpu.VMEM((2, 128), jnp.int32),
            pltpu.VMEM((2, 128), jnp.int32),
            pltpu.VMEM((NSLOT // 16,), jnp.int32),
            pltpu.VMEM((SPT,), jnp.int32),
            pltpu.VMEM((40, D_MODEL), jnp.float32),
            pltpu.VMEM((40, D_MODEL), jnp.float32),
            pltpu.VMEM_SHARED((NSLOT,), jnp.int32),
            pltpu.SemaphoreType.DMA,
            pltpu.SemaphoreType.DMA,
            pltpu.SemaphoreType.DMA,
            pltpu.SemaphoreType.DMA,
            pltpu.SemaphoreType.DMA,
        ],
    )(slotd_flat, xbf)


def _ffn_body(a_ref, w1_ref, b1_ref, w2_ref, b2_ref, y_ref, acc_ref):
    f = pl.program_id(1)
    a = a_ref[0]
    h = jnp.dot(a, w1_ref[0], preferred_element_type=jnp.float32)
    h = jnp.maximum(h + b1_ref[0], 0.0)
    part = jnp.dot(h, w2_ref[0], preferred_element_type=jnp.float32)

    @pl.when(f == 0)
    def _():
        acc_ref[...] = part

    @pl.when(f != 0)
    def _():
        acc_ref[...] = acc_ref[...] + part

    @pl.when(f == NFB - 1)
    def _():
        y_ref[0] = acc_ref[...] + b2_ref[0]


def _ffn(bufs_bf, w1b, b1, w2b, b2):
    return pl.pallas_call(
        _ffn_body,
        grid=(E, NFB),
        in_specs=[
            pl.BlockSpec((1, CAPACITY, D_MODEL), lambda e, f: (e, 0, 0)),
            pl.BlockSpec((1, D_MODEL, FBLK), lambda e, f: (e, 0, f)),
            pl.BlockSpec((1, 1, FBLK), lambda e, f: (e, 0, f)),
            pl.BlockSpec((1, FBLK, D_MODEL), lambda e, f: (e, f, 0)),
            pl.BlockSpec((1, 1, D_MODEL), lambda e, f: (e, 0, 0)),
        ],
        out_specs=pl.BlockSpec((1, CAPACITY, D_MODEL), lambda e, f: (e, 0, 0)),
        out_shape=jax.ShapeDtypeStruct((E, CAPACITY, D_MODEL), jnp.float32),
        scratch_shapes=[pltpu.VMEM((CAPACITY, D_MODEL), jnp.float32)],
    )(bufs_bf, w1b, b1, w2b, b2)


def _combine_body(slotc_hbm, gate_hbm, y_hbm, out_hbm,
                  slot_v, gate_v, rows_a, rows_b, out_a, out_b,
                  sem_ga, sem_gb, sem_wa, sem_wb):
    cid = lax.axis_index("c")
    sid = lax.axis_index("s")
    wid = sid * 2 + cid
    base_e = wid * TPT * TOP_K

    pltpu.sync_copy(slotc_hbm.at[pl.ds(base_e, TPT * TOP_K)], slot_v)
    pltpu.sync_copy(gate_hbm.at[pl.ds(base_e, TPT * TOP_K)],
                    gate_v.at[pl.ds(0, TPT * TOP_K)])

    rows = [rows_a, rows_b]
    outs = [out_a, out_b]
    sem_g = [sem_ga, sem_gb]
    sem_w = [sem_wa, sem_wb]
    NCH = 4

    gcp = [None] * NCH
    wcp = [None] * NCH
    gcp[0] = pltpu.async_copy(
        y_hbm.at[slot_v.at[pl.ds(0, 32)]], rows[0], sem_g[0])
    for k in range(NCH):
        p = k & 1
        gcp[k].wait()
        if k + 1 < NCH:
            gcp[k + 1] = pltpu.async_copy(
                y_hbm.at[slot_v.at[pl.ds((k + 1) * 32, 32)]],
                rows[1 - p], sem_g[1 - p])
        if k >= 2:
            wcp[k - 2].wait()
        rv = rows[p]
        ov = outs[p]

        def tok_i(i, _):
            gv = gate_v[pl.ds(k * 32 + 2 * i, 16)]
            g0 = gv[0]
            g1 = gv[1]

            for j in range(D_MODEL // 16):
                r0 = rv[2 * i, pl.ds(j * 16, 16)]
                r1 = rv[2 * i + 1, pl.ds(j * 16, 16)]
                ov[i, pl.ds(j * 16, 16)] = g0 * r0 + g1 * r1
            return 0
        lax.fori_loop(0, 16, tok_i, 0)
        wcp[k] = pltpu.async_copy(
            ov, out_hbm.at[pl.ds(wid * TPT + k * 16, 16)], sem_w[p])
    wcp[NCH - 2].wait()
    wcp[NCH - 1].wait()


def _combine(slotc_flat, gate_flat, y_flat):
    mesh = plsc.VectorSubcoreMesh(core_axis_name="c", subcore_axis_name="s", num_cores=2, num_subcores=16)
    return pl.kernel(
        _combine_body,
        out_type=jax.ShapeDtypeStruct((T, D_MODEL), jnp.float32),
        mesh=mesh,
        compiler_params=pltpu.CompilerParams(needs_layout_passes=False),
        scratch_types=[
            pltpu.VMEM((TPT * TOP_K,), jnp.int32),
            pltpu.VMEM((TPT * TOP_K + 32,), jnp.float32),
            pltpu.VMEM((32, D_MODEL), jnp.float32),
            pltpu.VMEM((32, D_MODEL), jnp.float32),
            pltpu.VMEM((16, D_MODEL), jnp.float32),
            pltpu.VMEM((16, D_MODEL), jnp.float32),
            pltpu.SemaphoreType.DMA,
            pltpu.SemaphoreType.DMA,
            pltpu.SemaphoreType.DMA,
            pltpu.SemaphoreType.DMA,
        ],
    )(slotc_flat, gate_flat, y_flat)


def kernel(x, Wg, w1, b1, w2, b2):
    slotd, slotc, gate = _router(x, Wg)
    buffers = _dispatch(slotd.reshape(-1), x)
    bufs = buffers.reshape(E, CAPACITY, D_MODEL)
    y = _ffn(bufs, w1, b1[:, None, :],
             w2, b2[:, None, :])
    out = _combine(slotc.reshape(-1), gate.reshape(-1),
                   y.reshape(NSLOT, D_MODEL))
    return out
